# Initial kernel scaffold; baseline (speedup 1.0000x reference)
import jax
import jax.numpy as jnp
from jax.experimental import pallas as pl


def kernel(user_id, time_stamp, user_table, ts_table, buckets, ts_mean, ts_std):
    raise NotImplementedError("write your pallas kernel here")


# speedup vs baseline: 1.9894x; 1.9894x over previous
import functools

import jax
import jax.numpy as jnp
from jax import lax
from jax.experimental import pallas as pl
from jax.experimental.pallas import tpu as pltpu
from jax.experimental.pallas import tpu_sc as plsc

_B = 16384
_D = 32
_NB = 1000
_NBP = 1024
_OW = 2 * _D + 1
_NC, _NS, _L = 2, 16, 16
_NW = _NC * _NS
_RPW = _B // _NW
_CH = 128
_NCH = _RPW // _CH
_STEPS = (512, 256, 128, 64, 32, 16, 8, 4, 2, 1)


def _body(uid_hbm, ts_hbm, utab_hbm, ttab_hbm, bkt_hbm, consts_hbm,
          out_hbm,
          uidx_v, bidx_v, ts_v, bkt_v, consts_v,
          urows_v, trows_v, out_v, sem_u, sem_t):
    wid = lax.axis_index("s") * _NC + lax.axis_index("c")
    base = wid * _RPW

    pltpu.sync_copy(bkt_hbm, bkt_v)
    pltpu.sync_copy(consts_hbm, consts_v)
    pltpu.sync_copy(ts_hbm.at[pl.ds(base, _RPW)], ts_v)
    for j in range(_NCH):
        pltpu.sync_copy(uid_hbm.at[pl.ds(base + j * _CH, _CH)], uidx_v.at[j])

    ucopies = [
        pltpu.async_copy(utab_hbm.at[uidx_v.at[j]],
                         urows_v.at[pl.ds(j * _CH, _CH)], sem_u)
        for j in range(_NCH)
    ]

    def search16(i, _):
        x = ts_v[pl.ds(i * _L, _L)]
        res = jnp.zeros((_L,), jnp.int32)
        for step in _STEPS:
            nxt = res + step
            b = plsc.load_gather(bkt_v, [nxt - 1])
            res = jnp.where(b <= x, nxt, res)
        bidx_v[i // (_CH // _L), pl.ds((i % (_CH // _L)) * _L, _L)] = res
        return _

    tcopies = []
    for j in range(_NCH):
        lax.fori_loop(j * (_CH // _L), (j + 1) * (_CH // _L), search16, 0,
                      unroll=False)
        tcopies.append(
            pltpu.async_copy(ttab_hbm.at[bidx_v.at[j]],
                             trows_v.at[pl.ds(j * _CH, _CH)], sem_t))

    mean = consts_v[pl.ds(0, _L)]
    std = consts_v[pl.ds(_L, _L)]
    lanes = lax.iota(jnp.int32, _L)

    for c in ucopies:
        c.wait()
    for c in tcopies:
        c.wait()

    def asm16(i, _):
        r0 = i * _L
        x = ts_v[pl.ds(r0, _L)]
        v = (x - mean) / std
        plsc.store_scatter(out_v, [(r0 + lanes) * _OW + (_OW - 1)], v)
        for rl in range(_L):
            r = r0 + rl
            o = r * _OW
            out_v[pl.ds(o, _L)] = urows_v[r, pl.ds(0, _L)]
            out_v[pl.ds(o + _L, _L)] = urows_v[r, pl.ds(_L, _L)]
            out_v[pl.ds(o + 2 * _L, _L)] = trows_v[r, pl.ds(0, _L)]
            out_v[pl.ds(o + 3 * _L, _L)] = trows_v[r, pl.ds(_L, _L)]
        return _

    lax.fori_loop(0, _RPW // _L, asm16, 0, unroll=False)
    pltpu.sync_copy(out_v, out_hbm.at[pl.ds(base * _OW, _RPW * _OW)])


@jax.jit
def _sc_call(uid, ts, utab, ttab, bkt_pad, consts):
    mesh = plsc.VectorSubcoreMesh(core_axis_name="c", subcore_axis_name="s")
    f = pl.kernel(
        _body,
        out_type=jax.ShapeDtypeStruct((_B * _OW,), jnp.float32),
        mesh=mesh,
        compiler_params=pltpu.CompilerParams(needs_layout_passes=False,
                                             use_tc_tiling_on_sc=False),
        scratch_types=[
            pltpu.VMEM((_NCH, _CH), jnp.int32),
            pltpu.VMEM((_NCH, _CH), jnp.int32),
            pltpu.VMEM((_RPW,), jnp.float32),
            pltpu.VMEM((_NBP,), jnp.float32),
            pltpu.VMEM((2 * _L,), jnp.float32),
            pltpu.VMEM((_RPW, _D), jnp.float32),
            pltpu.VMEM((_RPW, _D), jnp.float32),
            pltpu.VMEM((_RPW * _OW,), jnp.float32),
            pltpu.SemaphoreType.DMA,
            pltpu.SemaphoreType.DMA,
        ],
    )
    return f(uid, ts, utab, ttab, bkt_pad, consts)


def kernel(user_id, time_stamp, user_table, ts_table, buckets, ts_mean, ts_std):
    uid = user_id.astype(jnp.int32)
    nb = buckets.shape[0]
    bkt_pad = jnp.concatenate(
        [buckets.astype(jnp.float32),
         jnp.full((_NBP - nb,), jnp.inf, jnp.float32)])
    consts = jnp.concatenate(
        [jnp.full((_L,), ts_mean, jnp.float32),
         jnp.full((_L,), ts_std, jnp.float32)])
    out = _sc_call(uid, time_stamp.astype(jnp.float32),
                   user_table, ts_table, bkt_pad, consts)
    return out.reshape(_B, _OW)
